# baseline (speedup 1.0000x reference)
import jax
import jax.numpy as jnp
from jax import lax
from jax.experimental import pallas as pl
from jax.experimental.pallas import tpu as pltpu
import math

_N = 20000
_NP = 20480
_S = 2560
_C = 21
_IMG = 800.0
_SCORE_THRESH = 0.05
_NMS_THRESH = 0.5
_DETS = 100
_CLIP = math.log(1000.0 / 16.0)
_NEG = -1e10


def _nms_body(lt_ref, dx_ref, dy_ref, dw_ref, dh_ref, pt_ref,
              ob_ref, os_ref, ol_ref,
              cur_ref, x1_ref, y1_ref, x2_ref, y2_ref, a2_ref, cmx_ref):
    lt = lt_ref[...]
    m = jnp.max(lt, axis=0, keepdims=True)
    e = jnp.exp(lt - m)
    p = e / jnp.sum(e, axis=0, keepdims=True)

    x1r = pt_ref[0:1, :, :]
    y1r = pt_ref[1:2, :, :]
    x2r = pt_ref[2:3, :, :]
    y2r = pt_ref[3:4, :, :]
    wid = x2r - x1r
    hei = y2r - y1r
    cx = x1r + 0.5 * wid
    cy = y1r + 0.5 * hei
    dx = dx_ref[...] / 10.0
    dy = dy_ref[...] / 10.0
    dw = jnp.minimum(dw_ref[...] / 5.0, _CLIP)
    dh = jnp.minimum(dh_ref[...] / 5.0, _CLIP)
    pcx = dx * wid + cx
    pcy = dy * hei + cy
    pw = jnp.exp(dw) * wid
    ph = jnp.exp(dh) * hei
    x1 = jnp.clip(pcx - 0.5 * pw, 0.0, _IMG)
    y1 = jnp.clip(pcy - 0.5 * ph, 0.0, _IMG)
    x2 = jnp.clip(pcx + 0.5 * pw, 0.0, _IMG)
    y2 = jnp.clip(pcy + 0.5 * ph, 0.0, _IMG)

    rio = lax.broadcasted_iota(jnp.int32, (_C, 8, _S), 0)

    ws = x2 - x1
    hs = y2 - y1
    valid = (rio > 0) & (p > _SCORE_THRESH) & (ws >= 0.01) & (hs >= 0.01)
    cur0 = jnp.where(valid, p, _NEG)

    x1_ref[...] = x1
    y1_ref[...] = y1
    x2_ref[...] = x2
    y2_ref[...] = y2
    a2_ref[...] = ws * hs
    cur_ref[...] = cur0
    cmx_ref[...] = jnp.max(cur0, axis=0, keepdims=True)
    ob_ref[...] = jnp.zeros((_DETS, 4), jnp.float32)
    os_ref[...] = jnp.zeros((_DETS, 1), jnp.float32)
    ol_ref[...] = jnp.zeros((_DETS, 1), jnp.int32)

    ni3 = (lax.broadcasted_iota(jnp.int32, (1, 8, _S), 1) * _S +
           lax.broadcasted_iota(jnp.int32, (1, 8, _S), 2))
    ni_row = ni3
    sio_cb = lax.broadcasted_iota(jnp.int32, (_C, 8, 128), 1)
    lio_cb = lax.broadcasted_iota(jnp.int32, (_C, 8, 128), 2)
    rio_cb = lax.broadcasted_iota(jnp.int32, (_C, 8, 128), 0)
    l4 = lax.broadcasted_iota(jnp.int32, (1, 4), 1)
    big = jnp.int32(1 << 30)

    def cond(st):
        k, alive = st
        return alive & (k < _DETS)

    def body(st):
        k, alive = st
        cm = cmx_ref[...]
        gmax = jnp.max(cm)
        n = jnp.min(jnp.where(cm == gmax, ni3, big))
        s = n // _S
        l = n - s * _S
        g = pl.multiple_of((l // 128) * 128, 128)
        j = l - g
        col8 = cur_ref[:, :, pl.ds(g, 128)]
        pmc = (sio_cb == s) & (lio_cb == j)
        cmt = gmax
        ok = cmt > (_NEG / 2)
        c = jnp.min(jnp.where(pmc & (col8 == cmt), rio_cb, big))
        pm = pmc & (rio_cb == c)
        cx1 = jnp.max(jnp.where(pm, x1_ref[:, :, pl.ds(g, 128)], -jnp.inf))
        cy1 = jnp.max(jnp.where(pm, y1_ref[:, :, pl.ds(g, 128)], -jnp.inf))
        cx2 = jnp.max(jnp.where(pm, x2_ref[:, :, pl.ds(g, 128)], -jnp.inf))
        cy2 = jnp.max(jnp.where(pm, y2_ref[:, :, pl.ds(g, 128)], -jnp.inf))
        rx1 = x1_ref[pl.ds(c, 1), :, :]
        ry1 = y1_ref[pl.ds(c, 1), :, :]
        rx2 = x2_ref[pl.ds(c, 1), :, :]
        ry2 = y2_ref[pl.ds(c, 1), :, :]
        ra2 = a2_ref[pl.ds(c, 1), :, :]
        ix1 = jnp.maximum(cx1, rx1)
        iy1 = jnp.maximum(cy1, ry1)
        ix2 = jnp.minimum(cx2, rx2)
        iy2 = jnp.minimum(cy2, ry2)
        inter = jnp.maximum(ix2 - ix1, 0.0) * jnp.maximum(iy2 - iy1, 0.0)
        a1 = (cx2 - cx1) * (cy2 - cy1)
        union = a1 + ra2 - inter + 1e-9
        supp = inter > 0.5 * union
        rcur = cur_ref[pl.ds(c, 1), :, :]
        newr = jnp.where(supp | (ni_row == n), _NEG, rcur)
        cur_ref[pl.ds(c, 1), :, :] = newr
        cmx_ref[...] = jnp.max(cur_ref[...], axis=0, keepdims=True)
        okf = jnp.where(ok, 1.0, 0.0)
        row = jnp.where(l4 == 0, cx1,
              jnp.where(l4 == 1, cy1,
              jnp.where(l4 == 2, cx2, cy2))) * okf
        ob_ref[pl.ds(k, 1), :] = row
        os_ref[pl.ds(k, 1), :] = jnp.full((1, 1), jnp.where(ok, cmt, 0.0),
                                          dtype=jnp.float32)
        ol_ref[pl.ds(k, 1), :] = jnp.full((1, 1), jnp.where(ok, c, 0),
                                          dtype=jnp.int32)
        return jnp.where(ok, k + 1, k), ok

    lax.while_loop(cond, body, (jnp.int32(0), jnp.bool_(True)))


def kernel(class_logits, box_regression, proposals):
    pad = _NP - _N
    lt = jnp.pad(class_logits.T, ((0, 0), (0, pad))).reshape(_C, 8, _S)
    br = box_regression.reshape(_N, _C, 4)
    dxt = jnp.pad(br[:, :, 0].T, ((0, 0), (0, pad))).reshape(_C, 8, _S)
    dyt = jnp.pad(br[:, :, 1].T, ((0, 0), (0, pad))).reshape(_C, 8, _S)
    dwt = jnp.pad(br[:, :, 2].T, ((0, 0), (0, pad))).reshape(_C, 8, _S)
    dht = jnp.pad(br[:, :, 3].T, ((0, 0), (0, pad))).reshape(_C, 8, _S)
    pt = jnp.pad(proposals.T, ((0, 0), (0, pad))).reshape(4, 8, _S)
    ob, osc, olb = pl.pallas_call(
        _nms_body,
        out_shape=(
            jax.ShapeDtypeStruct((_DETS, 4), jnp.float32),
            jax.ShapeDtypeStruct((_DETS, 1), jnp.float32),
            jax.ShapeDtypeStruct((_DETS, 1), jnp.int32),
        ),
        scratch_shapes=[
            pltpu.VMEM((_C, 8, _S), jnp.float32),
            pltpu.VMEM((_C, 8, _S), jnp.float32),
            pltpu.VMEM((_C, 8, _S), jnp.float32),
            pltpu.VMEM((_C, 8, _S), jnp.float32),
            pltpu.VMEM((_C, 8, _S), jnp.float32),
            pltpu.VMEM((_C, 8, _S), jnp.float32),
            pltpu.VMEM((1, 8, _S), jnp.float32),
        ],
        compiler_params=pltpu.CompilerParams(
            vmem_limit_bytes=128 * 1024 * 1024,
        ),
    )(lt, dxt, dyt, dwt, dht, pt)
    return ob, osc.reshape(_DETS), olb.reshape(_DETS)
